# Initial kernel scaffold; baseline (speedup 1.0000x reference)
import jax
import jax.numpy as jnp
from jax.experimental import pallas as pl


def kernel(features, edge_index, W1l, b1l, W1r, W2l, b2l, W2r):
    raise NotImplementedError("write your pallas kernel here")


# speedup vs baseline: 4.2380x; 4.2380x over previous
import functools

import jax
import jax.numpy as jnp
from jax import lax
from jax.experimental import pallas as pl
from jax.experimental.pallas import tpu as pltpu
from jax.experimental.pallas import tpu_sc as plsc

_N = 10000
_E = 320000
_D = 128
_NC = 2
_NS = 16
_NW = _NC * _NS
_CHUNK = 128
_CPW = 79
_EP = _CHUNK * _CPW * _NW
_PN = 10240
_RPT = _PN // _NS
_BN = 1024


def _seg_body(with_deg, z_hbm, src_hbm, dst_hbm, *rest):
    if with_deg:
        (p0_hbm, p1_hbm, d0_hbm, d1_hbm,
         src_v, dst_v, rows_v, ones_v, zvec_v, acc_s, dacc_s, sem) = rest
    else:
        (p0_hbm, p1_hbm,
         src_v, dst_v, rows_v, acc_s, sem) = rest

    cid = lax.axis_index("c")
    sid = lax.axis_index("s")
    wid = cid * _NS + sid

    zero16 = jnp.zeros((16,), jnp.float32)

    def _zrow(i, c):
        rows_v[i // 8, pl.ds((i % 8) * 16, 16)] = zero16
        return c
    lax.fori_loop(0, _CHUNK * 8, _zrow, 0)

    if with_deg:
        one16 = jnp.ones((16,), jnp.float32)

        def _ofill(i, c):
            ones_v[pl.ds(i * 16, 16)] = one16
            return c
        lax.fori_loop(0, _CHUNK // 16, _ofill, 0)

        def _zfill(i, c):
            zvec_v[pl.ds(i * 16, 16)] = zero16
            return c
        lax.fori_loop(0, _RPT // 16, _zfill, 0)

    base = sid * _RPT
    for k in range(_RPT // _CHUNK):
        pltpu.sync_copy(rows_v, acc_s.at[pl.ds(base + k * _CHUNK, _CHUNK)])
    if with_deg:
        pltpu.sync_copy(zvec_v, dacc_s.at[pl.ds(base, _RPT)])
    plsc.subcore_barrier()

    cbase = wid * _CPW

    def _ebody(j, c):
        off = (cbase + j) * _CHUNK
        pltpu.sync_copy(src_hbm.at[pl.ds(off, _CHUNK)], src_v)
        pltpu.sync_copy(dst_hbm.at[pl.ds(off, _CHUNK)], dst_v)
        pltpu.async_copy(z_hbm.at[src_v], rows_v, sem).wait()
        pltpu.sync_copy(rows_v, acc_s.at[dst_v], add=True)
        if with_deg:
            pltpu.sync_copy(ones_v, dacc_s.at[dst_v], add=True)
        return c
    lax.fori_loop(0, _CPW, _ebody, 0)
    plsc.subcore_barrier()

    @pl.when(cid == 0)
    def _():
        pltpu.sync_copy(acc_s.at[pl.ds(base, _RPT)], p0_hbm.at[pl.ds(base, _RPT)])
        if with_deg:
            pltpu.sync_copy(dacc_s.at[pl.ds(base, _RPT)], d0_hbm.at[pl.ds(base, _RPT)])

    @pl.when(cid == 1)
    def _():
        pltpu.sync_copy(acc_s.at[pl.ds(base, _RPT)], p1_hbm.at[pl.ds(base, _RPT)])
        if with_deg:
            pltpu.sync_copy(dacc_s.at[pl.ds(base, _RPT)], d1_hbm.at[pl.ds(base, _RPT)])


def _make_seg_sum(with_deg):
    mesh = plsc.VectorSubcoreMesh(
        core_axis_name="c", subcore_axis_name="s",
        num_cores=_NC, num_subcores=_NS)
    out_type = [jax.ShapeDtypeStruct((_PN, _D), jnp.float32)] * 2
    scratch = [
        pltpu.VMEM((_CHUNK,), jnp.int32),
        pltpu.VMEM((_CHUNK,), jnp.int32),
        pltpu.VMEM((_CHUNK, _D), jnp.float32),
    ]
    if with_deg:
        out_type += [jax.ShapeDtypeStruct((_PN,), jnp.float32)] * 2
        scratch += [
            pltpu.VMEM((_CHUNK,), jnp.float32),
            pltpu.VMEM((_RPT,), jnp.float32),
        ]
    scratch.append(pltpu.MemorySpace.VMEM_SHARED((_PN, _D), jnp.float32))
    if with_deg:
        scratch.append(pltpu.MemorySpace.VMEM_SHARED((_PN,), jnp.float32))
    scratch.append(pltpu.SemaphoreType.DMA)
    return pl.kernel(
        functools.partial(_seg_body, with_deg),
        out_type=tuple(out_type),
        mesh=mesh,
        scratch_types=tuple(scratch),
    )


_seg_sum_deg = _make_seg_sum(True)
_seg_sum = _make_seg_sum(False)

_DOT = (((1,), (1,)), ((), ()))


def _mm_first_body(x_ref, wl_ref, wr_ref, b_ref, z_ref, r_ref):
    x = x_ref[...]
    z_ref[...] = lax.dot_general(x, wl_ref[...], _DOT,
                                 preferred_element_type=jnp.float32)
    r_ref[...] = lax.dot_general(x, wr_ref[...], _DOT,
                                 preferred_element_type=jnp.float32) + b_ref[...]


def _mm_mid_body(p0_ref, p1_ref, d0_ref, d1_ref, r1_ref, wl_ref, wr_ref,
                 b_ref, z_ref, r_ref):
    invd = 1.0 / jnp.maximum(d0_ref[...] + d1_ref[...], 1.0)
    h = jnp.maximum((p0_ref[...] + p1_ref[...]) * invd + r1_ref[...], 0.0)
    z_ref[...] = lax.dot_general(h, wl_ref[...], _DOT,
                                 preferred_element_type=jnp.float32)
    r_ref[...] = lax.dot_general(h, wr_ref[...], _DOT,
                                 preferred_element_type=jnp.float32) + b_ref[...]


def _fin_body(q0_ref, q1_ref, d0_ref, d1_ref, r2_ref, o_ref):
    invd = 1.0 / jnp.maximum(d0_ref[...] + d1_ref[...], 1.0)
    o_ref[...] = (q0_ref[...] + q1_ref[...]) * invd + r2_ref[...]


_ROWS = pl.BlockSpec((_BN, _D), lambda i: (i, 0))
_COL = pl.BlockSpec((_BN, 1), lambda i: (i, 0))
_W = pl.BlockSpec((_D, _D), lambda i: (0, 0))
_B = pl.BlockSpec((1, _D), lambda i: (0, 0))
_GRID = _PN // _BN

_mm_first = pl.pallas_call(
    _mm_first_body,
    grid=(_GRID,),
    in_specs=[_ROWS, _W, _W, _B],
    out_specs=[_ROWS, _ROWS],
    out_shape=[jax.ShapeDtypeStruct((_PN, _D), jnp.float32)] * 2,
)

_mm_mid = pl.pallas_call(
    _mm_mid_body,
    grid=(_GRID,),
    in_specs=[_ROWS, _ROWS, _COL, _COL, _ROWS, _W, _W, _B],
    out_specs=[_ROWS, _ROWS],
    out_shape=[jax.ShapeDtypeStruct((_PN, _D), jnp.float32)] * 2,
)

_fin = pl.pallas_call(
    _fin_body,
    grid=(_GRID,),
    in_specs=[_ROWS, _ROWS, _COL, _COL, _ROWS],
    out_specs=_ROWS,
    out_shape=jax.ShapeDtypeStruct((_PN, _D), jnp.float32),
)


def kernel(features, edge_index, W1l, b1l, W1r, W2l, b2l, W2r):
    src = edge_index[0]
    dst = edge_index[1]
    pad = _EP - _E
    srcp = jnp.concatenate([src, jnp.zeros((pad,), jnp.int32)])
    dstp = jnp.concatenate([dst, jnp.full((pad,), _N, jnp.int32)])
    b1r = b1l.reshape(1, _D)
    b2r = b2l.reshape(1, _D)

    z1, r1 = _mm_first(features, W1l, W1r, b1r)
    p0, p1, d0, d1 = _seg_sum_deg(z1, srcp, dstp)
    d0c = d0.reshape(_PN, 1)
    d1c = d1.reshape(_PN, 1)
    z2, r2 = _mm_mid(p0, p1, d0c, d1c, r1, W2l, W2r, b2r)
    q0, q1 = _seg_sum(z2, srcp, dstp)
    out = _fin(q0, q1, d0c, d1c, r2)
    return out[:_N]
